# Initial kernel scaffold; baseline (speedup 1.0000x reference)
import jax
import jax.numpy as jnp
from jax.experimental import pallas as pl


def kernel(x, table, W_out, b_out):
    raise NotImplementedError("write your pallas kernel here")


# speedup vs baseline: 8.2038x; 8.2038x over previous
import functools

import jax
import jax.numpy as jnp
from jax import lax
from jax.experimental import pallas as pl
from jax.experimental.pallas import tpu as pltpu
from jax.experimental.pallas import tpu_sc as plsc

NW = 32
CHUNK = 128
EMB = 32
NCH = 128


def _gather_body(x_hbm, table_hbm, out_hbm, idx_v, rows0, rows1, sem0, sem1):
    nchunks = x_hbm.shape[1]
    wid = lax.axis_index("c") * 16 + lax.axis_index("s")
    pltpu.sync_copy(x_hbm.at[wid], idx_v)
    rows = (rows0, rows1)
    sems = (sem0, sem1)
    pltpu.async_copy(table_hbm.at[idx_v.at[0]], rows0, sem0)

    def body(i, carry):
        for b in range(2):
            j = i * 2 + b

            @pl.when(j + 1 < nchunks)
            def _():
                pltpu.async_copy(
                    table_hbm.at[idx_v.at[j + 1]], rows[1 - b], sems[1 - b]
                )

            pltpu.make_async_copy(
                table_hbm.at[pl.ds(0, CHUNK)], rows[b], sems[b]
            ).wait()
            pltpu.sync_copy(rows[b], out_hbm.at[wid, j])
        return carry

    lax.fori_loop(0, nchunks // 2, body, 0)


def _mm_body(emb_ref, wt_ref, b_ref, out_ref):
    out_ref[...] = (
        jnp.dot(emb_ref[...], wt_ref[...], preferred_element_type=jnp.float32)
        + b_ref[...]
    )


def kernel(x, table, W_out, b_out):
    B, F = x.shape
    total = B * F
    per_w = total // NW
    nchunks = per_w // CHUNK

    x_r = x.reshape(NW, nchunks, CHUNK).astype(jnp.int32)

    gather = pl.kernel(
        _gather_body,
        out_type=jax.ShapeDtypeStruct((NW, nchunks, CHUNK, EMB), jnp.float32),
        mesh=plsc.VectorSubcoreMesh(core_axis_name="c", subcore_axis_name="s"),
        scratch_types=[
            pltpu.VMEM((nchunks, CHUNK), jnp.int32),
            pltpu.VMEM((CHUNK, EMB), jnp.float32),
            pltpu.VMEM((CHUNK, EMB), jnp.float32),
            pltpu.SemaphoreType.DMA,
            pltpu.SemaphoreType.DMA,
        ],
        compiler_params=pltpu.CompilerParams(use_tc_tiling_on_sc=False),
    )
    emb = gather(x_r, table).reshape(total, EMB)

    RB = 2048
    out = pl.pallas_call(
        _mm_body,
        grid=(total // RB,),
        in_specs=[
            pl.BlockSpec((RB, EMB), lambda i: (i, 0)),
            pl.BlockSpec((EMB, NCH), lambda i: (0, 0)),
            pl.BlockSpec((1, NCH), lambda i: (0, 0)),
        ],
        out_specs=pl.BlockSpec((RB, NCH), lambda i: (i, 0)),
        out_shape=jax.ShapeDtypeStruct((total, NCH), jnp.float32),
    )(emb, W_out.T, b_out.reshape(1, NCH))

    return out.reshape(B, F, NCH)


# speedup vs baseline: 14.3384x; 1.7478x over previous
import functools

import jax
import jax.numpy as jnp
from jax import lax
from jax.experimental import pallas as pl
from jax.experimental.pallas import tpu as pltpu
from jax.experimental.pallas import tpu_sc as plsc

NW = 32
CHUNK = 128
EMB = 32
NCH = 128
RB = 512


def _gather_body(x_hbm, table_hbm, out_hbm, idx_v, rows0, rows1, sem0, sem1):
    nchunks = x_hbm.shape[1]
    wid = lax.axis_index("c") * 16 + lax.axis_index("s")
    pltpu.sync_copy(x_hbm.at[wid], idx_v)
    rows = (rows0, rows1)
    sems = (sem0, sem1)
    pltpu.async_copy(table_hbm.at[idx_v.at[0]], rows0, sem0)

    def body(i, carry):
        for b in range(2):
            j = i * 2 + b

            @pl.when(j + 1 < nchunks)
            def _():
                pltpu.async_copy(
                    table_hbm.at[idx_v.at[j + 1]], rows[1 - b], sems[1 - b]
                )

            pltpu.make_async_copy(
                table_hbm.at[pl.ds(0, CHUNK)], rows[b], sems[b]
            ).wait()
            pltpu.sync_copy(rows[b], out_hbm.at[wid, j])
        return carry

    lax.fori_loop(0, nchunks // 2, body, 0)


def _mm_body(emb_ref, wt_ref, b_ref, out_ref):
    e = emb_ref[...]
    w = wt_ref[...]
    bias = b_ref[...]
    for k in range(4):
        out_ref[pl.ds(k * RB, RB), :] = (
            jnp.dot(
                e[:, k * EMB : (k + 1) * EMB],
                w,
                preferred_element_type=jnp.float32,
            )
            + bias
        )


def kernel(x, table, W_out, b_out):
    B, F = x.shape
    total = B * F
    per_w = total // NW
    nchunks = per_w // CHUNK

    ngroups = total // (4 * RB)
    x_r = (
        x.T.reshape(ngroups, 4, RB)
        .swapaxes(1, 2)
        .reshape(NW, nchunks, CHUNK)
        .astype(jnp.int32)
    )

    gather = pl.kernel(
        _gather_body,
        out_type=jax.ShapeDtypeStruct((NW, nchunks, CHUNK, EMB), jnp.float32),
        mesh=plsc.VectorSubcoreMesh(core_axis_name="c", subcore_axis_name="s"),
        scratch_types=[
            pltpu.VMEM((nchunks, CHUNK), jnp.int32),
            pltpu.VMEM((CHUNK, EMB), jnp.float32),
            pltpu.VMEM((CHUNK, EMB), jnp.float32),
            pltpu.SemaphoreType.DMA,
            pltpu.SemaphoreType.DMA,
        ],
        compiler_params=pltpu.CompilerParams(use_tc_tiling_on_sc=False),
    )
    emb128 = gather(x_r, table).reshape(total * EMB // 128, 128)

    out = pl.pallas_call(
        _mm_body,
        grid=(total * EMB // 128 // RB,),
        in_specs=[
            pl.BlockSpec((RB, 128), lambda i: (i, 0)),
            pl.BlockSpec((EMB, NCH), lambda i: (0, 0)),
            pl.BlockSpec((1, NCH), lambda i: (0, 0)),
        ],
        out_specs=pl.BlockSpec((4 * RB, NCH), lambda i: (i, 0)),
        out_shape=jax.ShapeDtypeStruct((total, NCH), jnp.float32),
    )(emb128, W_out.T, b_out.reshape(1, NCH))

    return out.reshape(F, B, NCH).transpose(1, 0, 2)


# speedup vs baseline: 18.9019x; 1.3183x over previous
import functools

import jax
import jax.numpy as jnp
from jax import lax
from jax.experimental import pallas as pl
from jax.experimental.pallas import tpu as pltpu
from jax.experimental.pallas import tpu_sc as plsc

NW = 32
CHUNK = 128
EMB = 32
NCH = 128
RB = 512


def _gather_body(x_hbm, table_hbm, out_hbm, idx_v, rows0, rows1, sem0, sem1):
    nchunks = x_hbm.shape[1]
    wid = lax.axis_index("c") * 16 + lax.axis_index("s")
    pltpu.sync_copy(x_hbm.at[wid], idx_v)
    rows = (rows0, rows1)
    sems = (sem0, sem1)
    pltpu.async_copy(table_hbm.at[idx_v.at[0]], rows0, sem0)

    def body(i, carry):
        for b in range(2):
            j = i * 2 + b

            @pl.when(j + 1 < nchunks)
            def _():
                pltpu.async_copy(
                    table_hbm.at[idx_v.at[j + 1]], rows[1 - b], sems[1 - b]
                )

            pltpu.make_async_copy(
                table_hbm.at[pl.ds(0, CHUNK)], rows[b], sems[b]
            ).wait()
            pltpu.sync_copy(rows[b], out_hbm.at[wid, j])
        return carry

    lax.fori_loop(0, nchunks // 2, body, 0)


CBR = 4096


def _pad_body(tt_ref, eye_ref, out_ref):
    t = tt_ref[...]
    eye = eye_ref[...]
    for k in range(4):
        out_ref[:, k * EMB : (k + 1) * EMB] = jax.lax.dot_general(
            t[:, k * CBR : (k + 1) * CBR],
            eye,
            (((0,), (0,)), ((), ())),
            preferred_element_type=jnp.float32,
        )


def _mm_body(emb_ref, wt_ref, b_ref, out_ref):
    e = emb_ref[...]
    w = wt_ref[...]
    bias = b_ref[...]
    for k in range(4):
        out_ref[pl.ds(k * RB, RB), :] = (
            jnp.dot(
                e[:, k * EMB : (k + 1) * EMB],
                w,
                preferred_element_type=jnp.float32,
            )
            + bias
        )


def kernel(x, table, W_out, b_out):
    B, F = x.shape
    total = B * F
    per_w = total // NW
    nchunks = per_w // CHUNK

    B4 = 4 * CBR
    xi = x.astype(jnp.int32)
    xp = 4 * ((xi // B4) * CBR + (xi % B4) % CBR) + (xi % B4) // CBR

    ngroups = total // (4 * RB)
    x_r = (
        xp.T.reshape(ngroups, 4, RB)
        .swapaxes(1, 2)
        .reshape(NW, nchunks, CHUNK)
    )

    gather = pl.kernel(
        _gather_body,
        out_type=jax.ShapeDtypeStruct((NW, nchunks, CHUNK, EMB), jnp.float32),
        mesh=plsc.VectorSubcoreMesh(core_axis_name="c", subcore_axis_name="s"),
        scratch_types=[
            pltpu.VMEM((nchunks, CHUNK), jnp.int32),
            pltpu.VMEM((CHUNK, EMB), jnp.float32),
            pltpu.VMEM((CHUNK, EMB), jnp.float32),
            pltpu.SemaphoreType.DMA,
            pltpu.SemaphoreType.DMA,
        ],
        compiler_params=pltpu.CompilerParams(use_tc_tiling_on_sc=False),
    )
    V = table.shape[0]
    tpad = pl.pallas_call(
        _pad_body,
        grid=(pl.cdiv(V, B4),),
        in_specs=[
            pl.BlockSpec((EMB, B4), lambda i: (0, i)),
            pl.BlockSpec((EMB, EMB), lambda i: (0, 0)),
        ],
        out_specs=pl.BlockSpec((CBR, 128), lambda i: (i, 0)),
        out_shape=jax.ShapeDtypeStruct((V // 4, 128), jnp.float32),
    )(table.T, jnp.eye(EMB, dtype=jnp.float32))
    table_lin = tpad.reshape(V, EMB)
    emb128 = gather(x_r, table_lin).reshape(total * EMB // 128, 128)

    out = pl.pallas_call(
        _mm_body,
        grid=(total * EMB // 128 // RB,),
        in_specs=[
            pl.BlockSpec((RB, 128), lambda i: (i, 0)),
            pl.BlockSpec((EMB, NCH), lambda i: (0, 0)),
            pl.BlockSpec((1, NCH), lambda i: (0, 0)),
        ],
        out_specs=pl.BlockSpec((4 * RB, NCH), lambda i: (i, 0)),
        out_shape=jax.ShapeDtypeStruct((total, NCH), jnp.float32),
    )(emb128, W_out.T, b_out.reshape(1, NCH))

    return out.reshape(F, B, NCH).transpose(1, 0, 2)
